# baseline (speedup 1.0000x reference)
import functools

import jax
import jax.numpy as jnp
from jax import lax
from jax.experimental import pallas as pl
from jax.experimental.pallas import tpu as pltpu
from jax.experimental.pallas import tpu_sc as plsc

B = 16384
F = 26
NC, NS, L = 2, 16, 16
NW = NC * NS
RPW = B // NW
GPW = RPW * F


def _body(x_hbm, table_hbm, bias_hbm, out_hbm,
          idx2d_v, idx_sh, idx_v, rows_v, acc_v, bias_v, sem):
    sid = lax.axis_index("s")
    wid = sid * NC + lax.axis_index("c")
    base = wid * RPW
    pltpu.sync_copy(x_hbm.at[pl.ds(base, RPW), :], idx2d_v)
    pltpu.sync_copy(bias_hbm, bias_v)
    for f in range(F):
        pltpu.sync_copy(idx2d_v.at[:, f], idx_sh.at[sid, pl.ds(f * RPW, RPW)])
    pltpu.sync_copy(idx_sh.at[sid], idx_v)
    pltpu.async_copy(table_hbm.at[idx_v], rows_v, sem).wait()

    def red(i, carry):
        b16 = pl.multiple_of(i * L, L)
        acc = bias_v[...]
        for f in range(F):
            acc = acc + rows_v[pl.ds(f * RPW + b16, L)]
        acc_v[pl.ds(b16, L)] = acc
        return carry

    lax.fori_loop(0, RPW // L, red, 0)
    pltpu.sync_copy(acc_v, out_hbm.at[pl.ds(base, RPW)])


_launch = functools.partial(
    pl.kernel,
    out_type=jax.ShapeDtypeStruct((B,), jnp.float32),
    mesh=plsc.VectorSubcoreMesh(
        core_axis_name="c", subcore_axis_name="s", num_cores=NC, num_subcores=NS),
    scratch_types=[
        pltpu.VMEM((RPW, F), jnp.int32),
        pltpu.VMEM_SHARED((NS, GPW), jnp.int32),
        pltpu.VMEM((GPW,), jnp.int32),
        pltpu.VMEM((GPW,), jnp.float32),
        pltpu.VMEM((RPW,), jnp.float32),
        pltpu.VMEM((L,), jnp.float32),
        pltpu.SemaphoreType.DMA,
    ],
)(_body)


def kernel(X, table, bias):
    tf = table.reshape(-1)
    b16 = jnp.broadcast_to(bias, (L,))
    out = _launch(X, tf, b16)
    return out.reshape(B, 1)


# speedup vs baseline: 1.5821x; 1.5821x over previous
import functools

import jax
import jax.numpy as jnp
from jax import lax
from jax.experimental import pallas as pl
from jax.experimental.pallas import tpu as pltpu
from jax.experimental.pallas import tpu_sc as plsc

B = 16384
F = 26
FH = F // 2
NC, NS, L = 2, 16, 16
NW = NC * NS
RPW = B // NW
GPW = RPW * F
HPW = RPW * FH


def _body(xf_hbm, table_hbm, bias_hbm, out_hbm,
          idx_v, rows_v, acc_v, bias_v, semA, semB):
    wid = lax.axis_index("s") * NC + lax.axis_index("c")
    base = wid * RPW
    pltpu.sync_copy(xf_hbm.at[pl.ds(wid * GPW, GPW)], idx_v)
    pltpu.sync_copy(bias_hbm, bias_v)
    cpA = pltpu.async_copy(table_hbm.at[idx_v.at[pl.ds(0, HPW)]],
                           rows_v.at[pl.ds(0, HPW)], semA)
    cpB = pltpu.async_copy(table_hbm.at[idx_v.at[pl.ds(HPW, HPW)]],
                           rows_v.at[pl.ds(HPW, HPW)], semB)
    cpA.wait()

    def red(flo, fhi):
        def step(i, carry):
            b16 = pl.multiple_of(i * L, L)
            acc = acc_v[pl.ds(b16, L)] if flo else bias_v[...]
            for f in range(flo, fhi):
                acc = acc + rows_v[pl.ds(f * RPW + b16, L)]
            acc_v[pl.ds(b16, L)] = acc
            return carry
        lax.fori_loop(0, RPW // L, step, 0)

    red(0, FH)
    cpB.wait()
    red(FH, F)
    pltpu.sync_copy(acc_v, out_hbm.at[pl.ds(base, RPW)])


_launch = functools.partial(
    pl.kernel,
    out_type=jax.ShapeDtypeStruct((B,), jnp.float32),
    mesh=plsc.VectorSubcoreMesh(
        core_axis_name="c", subcore_axis_name="s", num_cores=NC, num_subcores=NS),
    scratch_types=[
        pltpu.VMEM((GPW,), jnp.int32),
        pltpu.VMEM((GPW,), jnp.float32),
        pltpu.VMEM((RPW,), jnp.float32),
        pltpu.VMEM((L,), jnp.float32),
        pltpu.SemaphoreType.DMA,
        pltpu.SemaphoreType.DMA,
    ],
)(_body)


def kernel(X, table, bias):
    xf = X.reshape(NW, RPW, F).transpose(0, 2, 1).reshape(-1)
    tf = table.reshape(-1)
    b16 = jnp.broadcast_to(bias, (L,))
    out = _launch(xf, tf, b16)
    return out.reshape(B, 1)


# speedup vs baseline: 3.2175x; 2.0337x over previous
import functools

import jax
import jax.numpy as jnp
from jax import lax
from jax.experimental import pallas as pl
from jax.experimental.pallas import tpu as pltpu
from jax.experimental.pallas import tpu_sc as plsc

B = 16384
F = 26
V = 1000000
NC, NS, L = 2, 16, 16
NW = NC * NS
RPW = B // NW
GPW = RPW * F


def _body(xf_hbm, table_hbm, bias_hbm, out_hbm, idx_v, rows_v, acc_v, bias_v, sem):
    wid = lax.axis_index("s") * NC + lax.axis_index("c")
    base = wid * RPW
    pltpu.sync_copy(xf_hbm.at[pl.ds(wid * GPW, GPW)], idx_v)
    pltpu.sync_copy(bias_hbm, bias_v)
    tview = table_hbm.at[0]
    pltpu.async_copy(tview.at[idx_v], rows_v, sem).wait()

    def red(i, carry):
        b16 = pl.multiple_of(i * L, L)
        acc = bias_v[...]
        for f in range(F):
            acc = acc + rows_v[pl.ds(f * RPW + b16, L)]
        acc_v[pl.ds(b16, L)] = acc
        return carry

    lax.fori_loop(0, RPW // L, red, 0)
    pltpu.sync_copy(acc_v, out_hbm.at[pl.ds(base, RPW)])


_launch = functools.partial(
    pl.kernel,
    out_type=jax.ShapeDtypeStruct((B,), jnp.float32),
    mesh=plsc.VectorSubcoreMesh(
        core_axis_name="c", subcore_axis_name="s", num_cores=NC, num_subcores=NS),
    scratch_types=[
        pltpu.VMEM((GPW,), jnp.int32),
        pltpu.VMEM((GPW,), jnp.float32),
        pltpu.VMEM((RPW,), jnp.float32),
        pltpu.VMEM((L,), jnp.float32),
        pltpu.SemaphoreType.DMA,
    ],
)(_body)


def kernel(X, table, bias):
    xf = X.reshape(NW, RPW, F).transpose(0, 2, 1).reshape(-1)
    t1m = table.reshape(1, V)
    b16 = jnp.broadcast_to(bias, (L,))
    out = _launch(xf, t1m, b16)
    return out.reshape(B, 1)


# speedup vs baseline: 3.2656x; 1.0150x over previous
import functools

import jax
import jax.numpy as jnp
from jax import lax
from jax.experimental import pallas as pl
from jax.experimental.pallas import tpu as pltpu
from jax.experimental.pallas import tpu_sc as plsc

B = 16384
F = 26
V = 1000000
NC, NS, L = 2, 16, 16
NW = NC * NS
RPW = B // NW
GPW = RPW * F


def _body(xf_hbm, table_hbm, bias_hbm, out_hbm,
          idx_v, rows_v, acc_v, z_v, bias_v, sem):
    wid = lax.axis_index("s") * NC + lax.axis_index("c")
    base = wid * RPW
    pltpu.sync_copy(xf_hbm.at[pl.ds(wid * GPW, GPW)], idx_v)
    z_v[...] = lax.iota(jnp.int32, L) * 0
    cpb = pltpu.async_copy(bias_hbm.at[z_v], bias_v, sem)
    tview = table_hbm.at[0]
    cpg = pltpu.async_copy(tview.at[idx_v], rows_v, sem)
    cpb.wait()
    cpg.wait()
    bvec = bias_v[...]

    def red(i, carry):
        b16 = pl.multiple_of(i * L, L)
        acc = bvec
        for f in range(F):
            acc = acc + rows_v[pl.ds(f * RPW + b16, L)]
        acc_v[pl.ds(b16, L)] = acc
        return carry

    lax.fori_loop(0, RPW // L, red, 0)
    pltpu.sync_copy(acc_v, out_hbm.at[pl.ds(base, RPW)])


_launch = functools.partial(
    pl.kernel,
    out_type=jax.ShapeDtypeStruct((B,), jnp.float32),
    mesh=plsc.VectorSubcoreMesh(
        core_axis_name="c", subcore_axis_name="s", num_cores=NC, num_subcores=NS),
    scratch_types=[
        pltpu.VMEM((GPW,), jnp.int32),
        pltpu.VMEM((GPW,), jnp.float32),
        pltpu.VMEM((RPW,), jnp.float32),
        pltpu.VMEM((L,), jnp.int32),
        pltpu.VMEM((L,), jnp.float32),
        pltpu.SemaphoreType.DMA,
    ],
)(_body)


def kernel(X, table, bias):
    xf = X.reshape(NW, RPW, F).transpose(0, 2, 1).reshape(-1)
    t1m = table.reshape(1, V)
    out = _launch(xf, t1m, bias)
    return out.reshape(B, 1)
